# Initial kernel scaffold; baseline (speedup 1.0000x reference)
import jax
import jax.numpy as jnp
from jax.experimental import pallas as pl


def kernel(x, codebook):
    raise NotImplementedError("write your pallas kernel here")


# speedup vs baseline: 1.0940x; 1.0940x over previous
import functools

import jax
import jax.numpy as jnp
from jax.experimental import pallas as pl
from jax.experimental.pallas import tpu as pltpu


def _vq_body(x_ref, cb_ref, xhat_ref, onehot_ref, idx_ref):
    ncb = cb_ref.shape[0]
    for c in range(ncb):
        cbc = cb_ref[c]
        xc = x_ref[:, c, :]
        cnorm = jnp.sum(cbc * cbc, axis=-1)[None, :]
        xnorm = jnp.sum(xc * xc, axis=-1, keepdims=True)
        dot = jnp.dot(xc, cbc.T, preferred_element_type=jnp.float32)
        dist = (xnorm + cnorm) - 2.0 * dot
        idx = jnp.argmin(dist, axis=-1)
        iota = jax.lax.broadcasted_iota(jnp.int32, dist.shape, 1)
        oh = (iota == idx[:, None]).astype(jnp.float32)
        onehot_ref[:, c, :] = oh
        xhat_ref[:, c, :] = jnp.dot(oh, cbc, preferred_element_type=jnp.float32)
        idx_ref[:, c, :] = idx[:, None]


@functools.partial(jax.jit, static_argnames=("block_n",))
def _vq(x, codebook, block_n=128):
    n, ncb, dim = x.shape
    _, cb_size, _ = codebook.shape
    grid = (n // block_n,)
    out_shapes = (
        jax.ShapeDtypeStruct((n, ncb, dim), jnp.float32),
        jax.ShapeDtypeStruct((n, ncb, cb_size), jnp.float32),
        jax.ShapeDtypeStruct((n, ncb, 1), jnp.int32),
    )
    out_specs = (
        pl.BlockSpec((block_n, ncb, dim), lambda i: (i, 0, 0)),
        pl.BlockSpec((block_n, ncb, cb_size), lambda i: (i, 0, 0)),
        pl.BlockSpec((block_n, ncb, 1), lambda i: (i, 0, 0)),
    )
    in_specs = [
        pl.BlockSpec((block_n, ncb, dim), lambda i: (i, 0, 0)),
        pl.BlockSpec((ncb, cb_size, dim), lambda i: (0, 0, 0)),
    ]
    return pl.pallas_call(
        _vq_body,
        grid=grid,
        in_specs=in_specs,
        out_specs=out_specs,
        out_shape=out_shapes,
    )(x, codebook)


def kernel(x, codebook):
    return _vq(x, codebook)


# speedup vs baseline: 1.3124x; 1.1996x over previous
import functools

import jax
import jax.numpy as jnp
from jax.experimental import pallas as pl
from jax.experimental.pallas import tpu as pltpu


def _cnorm_body(cb_ref, cnorm_ref):
    cb_size = cb_ref.shape[1]
    chunk = 1024
    for k in range(0, cb_size, chunk):
        blk = cb_ref[0, k:k + chunk, :]
        cnorm_ref[0, 0, k:k + chunk] = jnp.sum(blk * blk, axis=-1)


def _vq_body(cnorm_ref, x_ref, cb_ref, xhat_ref, onehot_ref, idx_ref):
    ncb = cb_ref.shape[0]
    for c in range(ncb):
        cbc = cb_ref[c]
        xc = x_ref[:, c, :]
        cnorm = cnorm_ref[c, 0, :][None, :]
        xnorm = jnp.sum(xc * xc, axis=-1, keepdims=True)
        dot = jnp.dot(xc, cbc.T, preferred_element_type=jnp.float32)
        dist = (xnorm + cnorm) - 2.0 * dot
        idx = jnp.argmin(dist, axis=-1)
        iota = jax.lax.broadcasted_iota(jnp.int32, dist.shape, 1)
        oh = (iota == idx[:, None]).astype(jnp.float32)
        onehot_ref[:, c, :] = oh
        xhat_ref[:, c, :] = jnp.dot(oh, cbc, preferred_element_type=jnp.float32)
        idx_ref[:, c, :] = idx[:, None]


@functools.partial(jax.jit, static_argnames=("block_n",))
def _vq(x, codebook, block_n=128):
    n, ncb, dim = x.shape
    _, cb_size, _ = codebook.shape

    cnorm = pl.pallas_call(
        _cnorm_body,
        grid=(ncb,),
        in_specs=[pl.BlockSpec((1, cb_size, dim), lambda c: (c, 0, 0))],
        out_specs=pl.BlockSpec((1, 1, cb_size), lambda c: (c, 0, 0)),
        out_shape=jax.ShapeDtypeStruct((ncb, 1, cb_size), jnp.float32),
    )(codebook)

    grid = (n // block_n,)
    out_shapes = (
        jax.ShapeDtypeStruct((n, ncb, dim), jnp.float32),
        jax.ShapeDtypeStruct((n, ncb, cb_size), jnp.float32),
        jax.ShapeDtypeStruct((n, ncb, 1), jnp.int32),
    )
    out_specs = (
        pl.BlockSpec((block_n, ncb, dim), lambda i: (i, 0, 0)),
        pl.BlockSpec((block_n, ncb, cb_size), lambda i: (i, 0, 0)),
        pl.BlockSpec((block_n, ncb, 1), lambda i: (i, 0, 0)),
    )
    in_specs = [
        pl.BlockSpec((ncb, 1, cb_size), lambda i: (0, 0, 0)),
        pl.BlockSpec((block_n, ncb, dim), lambda i: (i, 0, 0)),
        pl.BlockSpec((ncb, cb_size, dim), lambda i: (0, 0, 0)),
    ]
    return pl.pallas_call(
        _vq_body,
        grid=grid,
        in_specs=in_specs,
        out_specs=out_specs,
        out_shape=out_shapes,
    )(cnorm, x, codebook)


def kernel(x, codebook):
    return _vq(x, codebook)
